# Initial kernel scaffold; baseline (speedup 1.0000x reference)
import jax
import jax.numpy as jnp
from jax.experimental import pallas as pl


def kernel(x, table, nw1, nb1, nw2, nb2, tw1, tb1, tw2, tb2, adm, gamma, beta):
    raise NotImplementedError("write your pallas kernel here")


# speedup vs baseline: 1.9339x; 1.9339x over previous
import functools

import jax
import jax.numpy as jnp
from jax import lax
from jax.experimental import pallas as pl
from jax.experimental.pallas import tpu as pltpu
from jax.experimental.pallas import tpu_sc as plsc

_T = 64
_J = 26
_ROWS = _T * _J
_KSUB = _ROWS // 128


def _sc_embedding_bag(table, idxc, n_tokens):
    info = plsc.get_sparse_core_info()
    nw = info.num_cores * info.num_subcores
    g_total = idxc.shape[0]
    cpw = g_total // nw
    d = table.shape[1]

    @functools.partial(
        pl.kernel,
        out_type=jax.ShapeDtypeStruct((n_tokens, d), jnp.float32),
        mesh=plsc.VectorSubcoreMesh(core_axis_name="c", subcore_axis_name="s"),
        compiler_params=pltpu.CompilerParams(use_tc_tiling_on_sc=False),
        scratch_types=[
            pltpu.VMEM((_KSUB, 128), jnp.int32),
            pltpu.VMEM((_ROWS, d), jnp.float32),
            pltpu.VMEM((_T, d), jnp.float32),
            pltpu.VMEM((_T + 16,), jnp.float32),
            pltpu.VMEM((1, d), jnp.float32),
            pltpu.SemaphoreType.DMA,
        ],
    )
    def k(table_h, idx_h, out_h, idx_v, rows_v, emb_v, cnt_v, row0_v, sem):
        wid = lax.axis_index("c") * info.num_subcores + lax.axis_index("s")
        pltpu.sync_copy(table_h.at[pl.ds(0, 1)], row0_v)

        def chunk_body(c, carry):
            g = wid * cpw + c
            pltpu.sync_copy(idx_h.at[g], idx_v)
            copies = [
                pltpu.async_copy(
                    table_h.at[idx_v.at[kk]],
                    rows_v.at[pl.ds(kk * 128, 128)],
                    sem,
                )
                for kk in range(_KSUB)
            ]
            for g16 in range(_T // 16):
                acc = jnp.zeros((16,), jnp.float32)
                for j in range(_J):
                    flat = j * _T + g16 * 16
                    v = idx_v[flat // 128, pl.ds(flat % 128, 16)]
                    acc = acc + jnp.where(v == 0, 1.0, 0.0)
                cnt_v[pl.ds(g16 * 16, 16)] = acc
            for cp in copies:
                cp.wait()

            def tok_body(t, carry2):
                cnt = cnt_v[pl.ds(t, 16)][0]
                for q in range(d // 16):
                    a = rows_v[t, pl.ds(q * 16, 16)]
                    for j in range(1, _J):
                        a = a + rows_v[j * _T + t, pl.ds(q * 16, 16)]
                    emb_v[t, pl.ds(q * 16, 16)] = (
                        a - cnt * row0_v[0, pl.ds(q * 16, 16)]
                    )
                return carry2

            lax.fori_loop(0, _T, tok_body, 0)
            pltpu.sync_copy(emb_v, out_h.at[pl.ds(g * _T, _T)])
            return carry

        lax.fori_loop(0, cpw, chunk_body, 0)

    return k(table, idxc)


def _tc_dense(xr, emb, nw1, nb1, nw2, nb2, tw1, tb1, tw2, tb2, admv, gv, bv):
    n, c = xr.shape
    d = emb.shape[1]
    bt = 1024
    grid = n // bt

    def body(x_ref, e_ref, nw1_r, nb1_r, nw2_r, nb2_r, tw1_r, tb1_r,
             tw2_r, tb2_r, adm_r, g_r, b_r, o_ref):
        xs = x_ref[...]
        num = xs[:, 5:6]
        nmask = jnp.isnan(num)
        numc = jnp.where(nmask, 0.0, num)
        h1 = jnp.maximum(numc * nw1_r[...] + nb1_r[...], 0.0)
        no = jnp.dot(h1, nw2_r[...], preferred_element_type=jnp.float32)
        no = jnp.where(nmask, 0.0, no + nb2_r[...])

        td = xs[:, 0:5]
        tmask = jnp.isnan(td[:, 0:1])
        tdc = jnp.where(jnp.isnan(td), 0.0, td)
        acc = tb1_r[...]
        for kk in range(5):
            acc = acc + tdc[:, kk:kk + 1] * tw1_r[kk:kk + 1, :]
        h2 = jnp.maximum(acc, 0.0)
        to = jnp.dot(h2, tw2_r[...], preferred_element_type=jnp.float32)
        to = jnp.where(tmask, 0.0, to + tb2_r[...])

        enc = e_ref[...] + no + to
        nrm = jnp.sqrt(jnp.sum(enc * enc, axis=1, keepdims=True))
        enc = enc / jnp.maximum(nrm, 1e-10)
        enc = enc + xs[:, 32:33] * adm_r[...]
        mu = jnp.mean(enc, axis=1, keepdims=True)
        dev = enc - mu
        var = jnp.mean(dev * dev, axis=1, keepdims=True)
        o_ref[...] = dev * lax.rsqrt(var + 1e-5) * g_r[...] + b_r[...]

    full = lambda shape: pl.BlockSpec(shape, lambda i: (0, 0))
    return pl.pallas_call(
        body,
        grid=(grid,),
        in_specs=[
            pl.BlockSpec((bt, c), lambda i: (i, 0)),
            pl.BlockSpec((bt, d), lambda i: (i, 0)),
            full((1, d)), full((1, d)), full((d, d)), full((1, d)),
            full((5, d)), full((1, d)), full((d, d)), full((1, d)),
            full((1, d)), full((1, d)), full((1, d)),
        ],
        out_specs=pl.BlockSpec((bt, d), lambda i: (i, 0)),
        out_shape=jax.ShapeDtypeStruct((n, d), jnp.float32),
    )(xr, emb, nw1, nb1, nw2, nb2, tw1, tb1, tw2, tb2, admv, gv, bv)


def kernel(x, table, nw1, nb1, nw2, nb2, tw1, tb1, tw2, tb2, adm, gamma, beta):
    b, s, c = x.shape
    n = b * s
    d = table.shape[1]
    categ = x[:, :, 6:32].astype(jnp.int32).reshape(n, _J)
    g_total = n // _T
    idxc = (categ.reshape(g_total, _T, _J)
            .transpose(0, 2, 1)
            .reshape(g_total, _KSUB, 128))
    emb = _sc_embedding_bag(table, idxc, n)
    out = _tc_dense(
        x.reshape(n, c), emb,
        nw1, nb1.reshape(1, d), nw2, nb2.reshape(1, d),
        tw1, tb1.reshape(1, d), tw2, tb2.reshape(1, d),
        adm.reshape(1, d), gamma.reshape(1, d), beta.reshape(1, d),
    )
    return out.reshape(b, s, d)
